# Initial kernel scaffold; baseline (speedup 1.0000x reference)
import jax
import jax.numpy as jnp
from jax.experimental import pallas as pl


def kernel(projection_head, sub_edge, edge2index, edge_embedding, W, b):
    raise NotImplementedError("write your pallas kernel here")


# speedup vs baseline: 3.3370x; 3.3370x over previous
import functools

import jax
import jax.numpy as jnp
from jax import lax
from jax.experimental import pallas as pl
from jax.experimental.pallas import tpu as pltpu
from jax.experimental.pallas import tpu_sc as plsc

NODE_NUM = 10000
NHID = 128
E_SUB = 320000
LANES = 16

NC = 2
NS = 16
NW = NC * NS
EPW = E_SUB // NW
CHUNK = 80
NCHUNK = EPW // CHUNK
NODE_PAD = 10240
ROWS_PER_TILE = NODE_PAD // NS


def _sc_aggregate(edge_emb, ph, e2i_r, src_r, dst_r, zeros):
    mesh = plsc.VectorSubcoreMesh(core_axis_name="c", subcore_axis_name="s")

    @functools.partial(
        pl.kernel,
        out_type=jax.ShapeDtypeStruct((NC, NODE_PAD, NHID), jnp.float32),
        mesh=mesh,
        scratch_types=[
            pltpu.VMEM((CHUNK,), jnp.int32),
            pltpu.VMEM((CHUNK,), jnp.int32),
            pltpu.VMEM((CHUNK,), jnp.int32),
            pltpu.VMEM((CHUNK, NHID), jnp.float32),
            pltpu.VMEM((CHUNK, NHID), jnp.float32),
            pltpu.VMEM_SHARED((NODE_PAD, NHID), jnp.float32),
            pltpu.SemaphoreType.DMA,
            pltpu.SemaphoreType.DMA,
        ],
    )
    def k(edge_emb_h, ph_h, e2i_h, src_h, dst_h, zeros_h, out_h,
          e2i_v, src_v, dst_v, ef_v, nf_v, acc, sem0, sem1):
        c = lax.axis_index("c")
        s = lax.axis_index("s")
        wid = s * NC + c

        rows = pl.ds(s * ROWS_PER_TILE, ROWS_PER_TILE)
        pltpu.sync_copy(zeros_h.at[rows], acc.at[rows])
        plsc.subcore_barrier()

        def chunk_body(i, carry):
            pltpu.sync_copy(e2i_h.at[wid, i], e2i_v)
            pltpu.sync_copy(src_h.at[wid, i], src_v)
            pltpu.sync_copy(dst_h.at[wid, i], dst_v)
            g_e = pltpu.async_copy(edge_emb_h.at[e2i_v], ef_v, sem0)
            g_n = pltpu.async_copy(ph_h.at[src_v], nf_v, sem1)
            g_e.wait()
            g_n.wait()

            def row_body(r, rc):
                for j in range(NHID // LANES):
                    sl = pl.ds(j * LANES, LANES)
                    ef_v[r, sl] = ef_v[r, sl] * nf_v[r, sl]
                return rc

            lax.fori_loop(0, CHUNK, row_body, 0)
            pltpu.sync_copy(ef_v, acc.at[dst_v], add=True)
            return carry

        lax.fori_loop(0, NCHUNK, chunk_body, 0)

        plsc.subcore_barrier()
        pltpu.sync_copy(acc.at[rows], out_h.at[c, rows])

    return k(edge_emb, ph, e2i_r, src_r, dst_r, zeros)


BLK = 1000


def _tc_update(a0, a1, ph, W, b2):
    def body(a0_ref, a1_ref, ph_ref, w_ref, b_ref, o_ref):
        x = a0_ref[...] + a1_ref[...]
        h = jnp.dot(x, w_ref[...], preferred_element_type=jnp.float32)
        h = jnp.maximum(h + b_ref[...], 0.0)
        o_ref[...] = h + ph_ref[...]

    return pl.pallas_call(
        body,
        grid=(NODE_NUM // BLK,),
        in_specs=[
            pl.BlockSpec((BLK, NHID), lambda i: (i, 0)),
            pl.BlockSpec((BLK, NHID), lambda i: (i, 0)),
            pl.BlockSpec((BLK, NHID), lambda i: (i, 0)),
            pl.BlockSpec((NHID, NHID), lambda i: (0, 0)),
            pl.BlockSpec((1, NHID), lambda i: (0, 0)),
        ],
        out_specs=pl.BlockSpec((BLK, NHID), lambda i: (i, 0)),
        out_shape=jax.ShapeDtypeStruct((NODE_NUM, NHID), jnp.float32),
    )(a0, a1, ph, W, b2)


def kernel(projection_head, sub_edge, edge2index, edge_embedding, W, b):
    src = sub_edge[0].reshape(NW, NCHUNK, CHUNK)
    dst = sub_edge[1].reshape(NW, NCHUNK, CHUNK)
    e2i = edge2index.reshape(NW, NCHUNK, CHUNK)
    zeros = jnp.zeros((NODE_PAD, NHID), jnp.float32)
    acc2 = _sc_aggregate(edge_embedding, projection_head, e2i, src, dst, zeros)
    return _tc_update(acc2[0], acc2[1], projection_head, W,
                      b.reshape(1, NHID))


# speedup vs baseline: 5.3011x; 1.5886x over previous
import functools

import jax
import jax.numpy as jnp
from jax import lax
from jax.experimental import pallas as pl
from jax.experimental.pallas import tpu as pltpu
from jax.experimental.pallas import tpu_sc as plsc

NODE_NUM = 10000
NHID = 128
E_SUB = 320000
LANES = 16

NC = 2
NS = 16
NW = NC * NS
EPW = E_SUB // NW
CHUNK = 40
NPAIR = EPW // (2 * CHUNK)
NODE_PAD = 10112
ROWS_PER_TILE = NODE_PAD // NS


def _sc_aggregate(edge_emb, ph, idx_r, zeros):
    mesh = plsc.VectorSubcoreMesh(core_axis_name="c", subcore_axis_name="s")

    @functools.partial(
        pl.kernel,
        out_type=jax.ShapeDtypeStruct((NC, NODE_PAD, NHID), jnp.float32),
        mesh=mesh,
        scratch_types=[
            pltpu.VMEM((6, CHUNK), jnp.int32),
            pltpu.VMEM((6, CHUNK), jnp.int32),
            pltpu.VMEM((CHUNK, NHID), jnp.float32),
            pltpu.VMEM((CHUNK, NHID), jnp.float32),
            pltpu.VMEM((CHUNK, NHID), jnp.float32),
            pltpu.VMEM((CHUNK, NHID), jnp.float32),
            pltpu.VMEM_SHARED((NODE_PAD, NHID), jnp.float32),
            pltpu.SemaphoreType.DMA,
            pltpu.SemaphoreType.DMA,
            pltpu.SemaphoreType.DMA,
            pltpu.SemaphoreType.DMA,
        ],
    )
    def k(edge_emb_h, ph_h, idx_h, zeros_h, out_h,
          i0, i1, ef0, ef1, nf0, nf1, acc, se0, se1, sn0, sn1):
        c = lax.axis_index("c")
        s = lax.axis_index("s")
        wid = s * NC + c

        efs = (ef0, ef1)
        nfs = (nf0, nf1)
        ses = (se0, se1)
        sns = (sn0, sn1)

        def issue(I, b):
            pltpu.async_copy(edge_emb_h.at[I.at[0 + b]], efs[b], ses[b])
            pltpu.async_copy(ph_h.at[I.at[2 + b]], nfs[b], sns[b])

        def wait(b):
            pltpu.make_async_copy(
                edge_emb_h.at[pl.ds(0, CHUNK)], efs[b], ses[b]).wait()
            pltpu.make_async_copy(
                ph_h.at[pl.ds(0, CHUNK)], nfs[b], sns[b]).wait()

        def compute(b):
            ef = efs[b]
            nf = nfs[b]

            def row_body(r, rc):
                for j in range(NHID // LANES):
                    sl = pl.ds(j * LANES, LANES)
                    nf[r, sl] = ef[r, sl] * nf[r, sl]
                return rc

            lax.fori_loop(0, CHUNK, row_body, 0)

        def scatter(I, b):
            pltpu.sync_copy(nfs[b], acc.at[I.at[4 + b]], add=True)

        def half(o, Icur, Inext):
            pltpu.sync_copy(idx_h.at[wid, o + 1], Inext)
            for b in (0, 1):
                wait(b)
                compute(b)
                scatter(Icur, b)
                issue(Inext, b)

        rows = pl.ds(s * ROWS_PER_TILE, ROWS_PER_TILE)
        pltpu.sync_copy(zeros_h.at[rows], acc.at[rows])
        plsc.subcore_barrier()

        pltpu.sync_copy(idx_h.at[wid, 0], i0)
        issue(i0, 0)
        issue(i0, 1)

        def outer(oo, carry):
            half(2 * oo, i0, i1)
            half(2 * oo + 1, i1, i0)
            return carry

        lax.fori_loop(0, (NPAIR - 1) // 2, outer, 0)

        for b in (0, 1):
            wait(b)
            compute(b)
            scatter(i0, b)

        plsc.subcore_barrier()
        pltpu.sync_copy(acc.at[rows], out_h.at[c, rows])

    return k(edge_emb, ph, idx_r, zeros)


BLK = 1000


def _tc_update(a0, a1, ph, W, b2):
    def body(a0_ref, a1_ref, ph_ref, w_ref, b_ref, o_ref):
        x = a0_ref[...] + a1_ref[...]
        h = jnp.dot(x, w_ref[...], preferred_element_type=jnp.float32)
        h = jnp.maximum(h + b_ref[...], 0.0)
        o_ref[...] = h + ph_ref[...]

    return pl.pallas_call(
        body,
        grid=(NODE_NUM // BLK,),
        in_specs=[
            pl.BlockSpec((BLK, NHID), lambda i: (i, 0)),
            pl.BlockSpec((BLK, NHID), lambda i: (i, 0)),
            pl.BlockSpec((BLK, NHID), lambda i: (i, 0)),
            pl.BlockSpec((NHID, NHID), lambda i: (0, 0)),
            pl.BlockSpec((1, NHID), lambda i: (0, 0)),
        ],
        out_specs=pl.BlockSpec((BLK, NHID), lambda i: (i, 0)),
        out_shape=jax.ShapeDtypeStruct((NODE_NUM, NHID), jnp.float32),
    )(a0, a1, ph, W, b2)


def kernel(projection_head, sub_edge, edge2index, edge_embedding, W, b):
    e2i = edge2index.reshape(NW, NPAIR, 2, CHUNK)
    src = sub_edge[0].reshape(NW, NPAIR, 2, CHUNK)
    dst = sub_edge[1].reshape(NW, NPAIR, 2, CHUNK)
    idx_r = jnp.concatenate([e2i, src, dst], axis=2)
    zeros = jnp.zeros((NODE_PAD, NHID), jnp.float32)
    acc2 = _sc_aggregate(edge_embedding, projection_head, idx_r, zeros)
    return _tc_update(acc2[0], acc2[1], projection_head, W,
                      b.reshape(1, NHID))
